# baseline (speedup 1.0000x reference)
import functools

import jax
import jax.numpy as jnp
from jax import lax
from jax.experimental import pallas as pl
from jax.experimental.pallas import tpu as pltpu
from jax.experimental.pallas import tpu_sc as plsc

T, D, F, E, K = 2048, 1024, 768, 64, 2
N = T * K
TILE = 64
P = 8192
P_TILES = P // TILE
NW = 32
CHUNK = 32
LANES = 16


def _router_body(x_ref, rk_ref, i1_ref, i2_ref, w1_ref, w2_ref, xbf_ref):
    xbf_ref[...] = x_ref[...].astype(jnp.bfloat16)
    s = jnp.dot(x_ref[...], rk_ref[...], preferred_element_type=jnp.float32)
    iota = lax.broadcasted_iota(jnp.int32, s.shape, 1)
    m1 = jnp.max(s, axis=1, keepdims=True)
    i1 = jnp.min(jnp.where(s == m1, iota, E), axis=1, keepdims=True)
    s2 = jnp.where(iota == i1, -jnp.inf, s)
    m2 = jnp.max(s2, axis=1, keepdims=True)
    i2 = jnp.min(jnp.where(s2 == m2, iota, E), axis=1, keepdims=True)
    w1 = 1.0 / (1.0 + jnp.exp(m2 - m1))
    i1_ref[...] = i1
    i2_ref[...] = i2
    w1_ref[...] = w1
    w2_ref[...] = 1.0 - w1


def _router(xf, rk):
    i1, i2, w1, w2, xbf = pl.pallas_call(
        _router_body,
        out_shape=[
            jax.ShapeDtypeStruct((T, 1), jnp.int32),
            jax.ShapeDtypeStruct((T, 1), jnp.int32),
            jax.ShapeDtypeStruct((T, 1), jnp.float32),
            jax.ShapeDtypeStruct((T, 1), jnp.float32),
            jax.ShapeDtypeStruct((T, D), jnp.bfloat16),
        ],
    )(xf, rk)
    return i1[:, 0], i2[:, 0], w1[:, 0], w2[:, 0], xbf


def _build_metadata(i1, i2, w1, w2):
    e_pairs = jnp.stack([i1, i2], axis=1).reshape(-1)
    w_pairs = jnp.stack([w1, w2], axis=1).reshape(-1)
    oh = (e_pairs[:, None] == jnp.arange(E)[None, :]).astype(jnp.int32)
    csum = jnp.cumsum(oh, axis=0)
    rank = jnp.sum(csum * oh, axis=1) - 1
    counts = csum[-1]
    padded = ((counts + TILE - 1) // TILE) * TILE
    starts = jnp.concatenate([jnp.zeros((1,), jnp.int32),
                              jnp.cumsum(padded)[:-1].astype(jnp.int32)])
    slot = (starts[e_pairs] + rank).astype(jnp.int32)
    tile_cum = jnp.cumsum(padded // TILE)
    tile_expert = jnp.minimum(
        jnp.searchsorted(tile_cum, jnp.arange(P_TILES), side="right"),
        E - 1).astype(jnp.int32)
    tok_of_pair = (jnp.arange(N, dtype=jnp.int32) // K)
    sorted_tok = jnp.zeros((P,), jnp.int32).at[slot].set(tok_of_pair)
    sorted_w = jnp.zeros((P,), jnp.float32).at[slot].set(w_pairs)
    sp = slot.reshape(T, K)
    return sorted_tok, sorted_w, tile_expert, sp[:, 0], sp[:, 1]


def _dispatch(xpacked, sorted_tok):
    mesh = plsc.VectorSubcoreMesh(core_axis_name="c", subcore_axis_name="s")
    rows_w = P // NW
    nch = rows_w // CHUNK
    nbuf = 3
    tok3 = sorted_tok.reshape(NW, nch, CHUNK)

    @functools.partial(
        pl.kernel,
        out_type=jax.ShapeDtypeStruct((P, D // 2), jnp.int32),
        mesh=mesh,
        scratch_types=[
            pltpu.VMEM((nch, CHUNK), jnp.int32),
            pltpu.VMEM((nbuf, CHUNK, D // 2), jnp.int32),
            pltpu.SemaphoreType.DMA,
            pltpu.SemaphoreType.DMA,
        ],
    )
    def k(tok_hbm, x_hbm, xs_hbm, idx_v, bufs, sem_g, sem_o):
        wid = lax.axis_index("s") * 2 + lax.axis_index("c")
        base = wid * rows_w
        pltpu.sync_copy(tok_hbm.at[wid], idx_v)
        for c in range(nbuf):
            pltpu.async_copy(x_hbm.at[idx_v.at[c]], bufs.at[c], sem_g)
        for c in range(nch):
            pltpu.make_async_copy(
                x_hbm.at[idx_v.at[c]], bufs.at[c % nbuf], sem_g).wait()
            pltpu.async_copy(
                bufs.at[c % nbuf],
                xs_hbm.at[pl.ds(base + c * CHUNK, CHUNK)], sem_o)
            if c + nbuf < nch:
                pltpu.make_async_copy(
                    bufs.at[c % nbuf],
                    xs_hbm.at[pl.ds(base + c * CHUNK, CHUNK)], sem_o).wait()
                pltpu.async_copy(
                    x_hbm.at[idx_v.at[c + nbuf]], bufs.at[c % nbuf], sem_g)
        for c in range(nch - nbuf, nch):
            pltpu.make_async_copy(
                bufs.at[c % nbuf],
                xs_hbm.at[pl.ds(base + c * CHUNK, CHUNK)], sem_o).wait()

    return k(tok3, xpacked)


def _ffn_body(te_ref, xs_ref, g_ref, u_ref, d_ref, sw_ref, ys_ref):
    xt = xs_ref[...].astype(jnp.float32)
    g = jnp.dot(xt, g_ref[0], preferred_element_type=jnp.float32)
    u = jnp.dot(xt, u_ref[0], preferred_element_type=jnp.float32)
    h = g * jax.nn.sigmoid(g) * u
    y = jnp.dot(h, d_ref[0], preferred_element_type=jnp.float32)
    ys_ref[...] = y * sw_ref[0, 0, :][:, None]


def _ffn(xs, gate_proj, up_proj, down_proj, sorted_w, tile_expert):
    sw3 = sorted_w.reshape(P_TILES, 1, TILE)
    grid_spec = pltpu.PrefetchScalarGridSpec(
        num_scalar_prefetch=1,
        grid=(P_TILES,),
        in_specs=[
            pl.BlockSpec((TILE, D), lambda i, te: (i, 0)),
            pl.BlockSpec((1, D, F), lambda i, te: (te[i], 0, 0)),
            pl.BlockSpec((1, D, F), lambda i, te: (te[i], 0, 0)),
            pl.BlockSpec((1, F, D), lambda i, te: (te[i], 0, 0)),
            pl.BlockSpec((1, 1, TILE), lambda i, te: (i, 0, 0)),
        ],
        out_specs=pl.BlockSpec((TILE, D), lambda i, te: (i, 0)),
    )
    return pl.pallas_call(
        _ffn_body,
        grid_spec=grid_spec,
        out_shape=jax.ShapeDtypeStruct((P, D), jnp.float32),
    )(tile_expert, xs, gate_proj, up_proj, down_proj, sw3)


def _combine(ys, s1, s2):
    mesh = plsc.VectorSubcoreMesh(core_axis_name="c", subcore_axis_name="s")
    tok_w = T // NW
    nch = tok_w // CHUNK
    nvec = CHUNK * (D // LANES)

    @functools.partial(
        pl.kernel,
        out_type=jax.ShapeDtypeStruct((T, D), jnp.float32),
        mesh=mesh,
        scratch_types=[
            pltpu.VMEM((CHUNK,), jnp.int32),
            pltpu.VMEM((CHUNK,), jnp.int32),
            pltpu.VMEM((CHUNK, D), jnp.float32),
            pltpu.VMEM((CHUNK, D), jnp.float32),
            pltpu.SemaphoreType.DMA,
        ],
    )
    def k(s1_hbm, s2_hbm, ys_hbm, out_hbm, i1_v, i2_v, y1_v, y2_v, sem):
        wid = lax.axis_index("s") * 2 + lax.axis_index("c")
        base = wid * tok_w

        def chunk_body(c, carry):
            b = base + c * CHUNK
            pltpu.sync_copy(s1_hbm.at[pl.ds(b, CHUNK)], i1_v)
            pltpu.sync_copy(s2_hbm.at[pl.ds(b, CHUNK)], i2_v)
            pltpu.async_copy(ys_hbm.at[i1_v], y1_v, sem).wait()
            pltpu.async_copy(ys_hbm.at[i2_v], y2_v, sem).wait()

            def add_body(i, carry2):
                r = i // (D // LANES)
                col = (i % (D // LANES)) * LANES
                a = y1_v[r, pl.ds(col, LANES)]
                bvec = y2_v[r, pl.ds(col, LANES)]
                y1_v[r, pl.ds(col, LANES)] = a + bvec
                return carry2

            lax.fori_loop(0, nvec, add_body, 0)
            pltpu.sync_copy(y1_v, out_hbm.at[pl.ds(b, CHUNK)])
            return carry

        lax.fori_loop(0, nch, chunk_body, 0)

    return k(s1, s2, ys)


def kernel(x, router_kernel, gate_proj, up_proj, down_proj):
    b, t, d = x.shape
    xf = x.reshape(t, d)
    i1, i2, w1, w2, xbf = _router(xf, router_kernel)
    sorted_tok, sorted_w, tile_expert, s1, s2 = _build_metadata(i1, i2, w1, w2)
    xpacked = lax.bitcast_convert_type(
        xbf.reshape(T, D // 2, 2), jnp.int32)
    xs_packed = _dispatch(xpacked, sorted_tok)
    xs = lax.bitcast_convert_type(xs_packed, jnp.bfloat16).reshape(P, D)
    ys = _ffn(xs, gate_proj, up_proj, down_proj,
              sorted_w, tile_expert)
    out = _combine(ys, s1, s2)
    return out.reshape(b, t, d)


# speedup vs baseline: 1.2847x; 1.2847x over previous
import functools

import jax
import jax.numpy as jnp
from jax import lax
from jax.experimental import pallas as pl
from jax.experimental.pallas import tpu as pltpu
from jax.experimental.pallas import tpu_sc as plsc

T, D, F, E, K = 2048, 1024, 768, 64, 2
N = T * K
TILE = 64
P = 8192
P_TILES = P // TILE
NW = 32
CHUNK = 32
LANES = 16


def _rtne_bf16_bits(v):
    bits = lax.bitcast_convert_type(v, jnp.int32)
    lsb = lax.shift_right_logical(bits, 16) & 1
    return (bits + 0x7FFF + lsb) & jnp.int32(-65536)


def _router_body(x_ref, rk_ref, i1_ref, i2_ref, w1_ref, w2_ref, xp_ref):
    xv = x_ref[...]
    ha = _rtne_bf16_bits(xv[:, : D // 2])
    hb = _rtne_bf16_bits(xv[:, D // 2:])
    xp_ref[...] = ha | lax.shift_right_logical(hb, 16)
    s = jnp.dot(x_ref[...], rk_ref[...], preferred_element_type=jnp.float32)
    iota = lax.broadcasted_iota(jnp.int32, s.shape, 1)
    m1 = jnp.max(s, axis=1, keepdims=True)
    i1 = jnp.min(jnp.where(s == m1, iota, E), axis=1, keepdims=True)
    s2 = jnp.where(iota == i1, -jnp.inf, s)
    m2 = jnp.max(s2, axis=1, keepdims=True)
    i2 = jnp.min(jnp.where(s2 == m2, iota, E), axis=1, keepdims=True)
    w1 = 1.0 / (1.0 + jnp.exp(m2 - m1))
    i1_ref[...] = i1
    i2_ref[...] = i2
    w1_ref[...] = w1
    w2_ref[...] = 1.0 - w1


def _router(xf, rk):
    i1, i2, w1, w2, xbf = pl.pallas_call(
        _router_body,
        out_shape=[
            jax.ShapeDtypeStruct((T, 1), jnp.int32),
            jax.ShapeDtypeStruct((T, 1), jnp.int32),
            jax.ShapeDtypeStruct((T, 1), jnp.float32),
            jax.ShapeDtypeStruct((T, 1), jnp.float32),
            jax.ShapeDtypeStruct((T, D // 2), jnp.int32),
        ],
    )(xf, rk)
    return i1[:, 0], i2[:, 0], w1[:, 0], w2[:, 0], xbf


def _build_metadata(i1, i2, w1, w2):
    e_pairs = jnp.stack([i1, i2], axis=1).reshape(-1)
    w_pairs = jnp.stack([w1, w2], axis=1).reshape(-1)
    oh = (e_pairs[:, None] == jnp.arange(E)[None, :]).astype(jnp.int32)
    csum = jnp.cumsum(oh, axis=0)
    rank = jnp.sum(csum * oh, axis=1) - 1
    counts = csum[-1]
    padded = ((counts + TILE - 1) // TILE) * TILE
    starts = jnp.concatenate([jnp.zeros((1,), jnp.int32),
                              jnp.cumsum(padded)[:-1].astype(jnp.int32)])
    slot = (starts[e_pairs] + rank).astype(jnp.int32)
    tile_cum = jnp.cumsum(padded // TILE)
    tile_expert = jnp.minimum(
        jnp.searchsorted(tile_cum, jnp.arange(P_TILES), side="right"),
        E - 1).astype(jnp.int32)
    tok_of_pair = (jnp.arange(N, dtype=jnp.int32) // K)
    sorted_tok = jnp.zeros((P,), jnp.int32).at[slot].set(tok_of_pair)
    sorted_w = jnp.zeros((P,), jnp.float32).at[slot].set(w_pairs)
    sp = slot.reshape(T, K)
    return sorted_tok, sorted_w, tile_expert, sp[:, 0], sp[:, 1]


def _dispatch(xpacked, sorted_tok):
    mesh = plsc.VectorSubcoreMesh(core_axis_name="c", subcore_axis_name="s")
    rows_w = P // NW
    nch = rows_w // CHUNK
    nbuf = 3
    tok3 = sorted_tok.reshape(NW, nch, CHUNK)

    @functools.partial(
        pl.kernel,
        out_type=jax.ShapeDtypeStruct((P, D // 2), jnp.int32),
        mesh=mesh,
        scratch_types=[
            pltpu.VMEM((nch, CHUNK), jnp.int32),
            pltpu.VMEM((nbuf, CHUNK, D // 2), jnp.int32),
            pltpu.SemaphoreType.DMA,
            pltpu.SemaphoreType.DMA,
        ],
    )
    def k(tok_hbm, x_hbm, xs_hbm, idx_v, bufs, sem_g, sem_o):
        wid = lax.axis_index("s") * 2 + lax.axis_index("c")
        base = wid * rows_w
        pltpu.sync_copy(tok_hbm.at[wid], idx_v)
        for c in range(nbuf):
            pltpu.async_copy(x_hbm.at[idx_v.at[c]], bufs.at[c], sem_g)
        for c in range(nch):
            pltpu.make_async_copy(
                x_hbm.at[idx_v.at[c]], bufs.at[c % nbuf], sem_g).wait()
            pltpu.async_copy(
                bufs.at[c % nbuf],
                xs_hbm.at[pl.ds(base + c * CHUNK, CHUNK)], sem_o)
            if c + nbuf < nch:
                pltpu.make_async_copy(
                    bufs.at[c % nbuf],
                    xs_hbm.at[pl.ds(base + c * CHUNK, CHUNK)], sem_o).wait()
                pltpu.async_copy(
                    x_hbm.at[idx_v.at[c + nbuf]], bufs.at[c % nbuf], sem_g)
        for c in range(nch - nbuf, nch):
            pltpu.make_async_copy(
                bufs.at[c % nbuf],
                xs_hbm.at[pl.ds(base + c * CHUNK, CHUNK)], sem_o).wait()

    return k(tok3, xpacked)


def _ffn_body(te_ref, xs_ref, g_ref, u_ref, d_ref, sw_ref, ys_ref):
    packed = xs_ref[...]
    xa = lax.bitcast_convert_type(packed & jnp.int32(-65536), jnp.float32)
    xb = lax.bitcast_convert_type(lax.shift_left(packed, 16), jnp.float32)
    xt = jnp.concatenate([xa, xb], axis=1)
    g = jnp.dot(xt, g_ref[0], preferred_element_type=jnp.float32)
    u = jnp.dot(xt, u_ref[0], preferred_element_type=jnp.float32)
    h = g * jax.nn.sigmoid(g) * u
    y = jnp.dot(h, d_ref[0], preferred_element_type=jnp.float32)
    ys_ref[...] = y * sw_ref[0, 0, :][:, None]


def _ffn(xs, gate_proj, up_proj, down_proj, sorted_w, tile_expert):
    sw3 = sorted_w.reshape(P_TILES, 1, TILE)
    grid_spec = pltpu.PrefetchScalarGridSpec(
        num_scalar_prefetch=1,
        grid=(P_TILES,),
        in_specs=[
            pl.BlockSpec((TILE, D // 2), lambda i, te: (i, 0)),
            pl.BlockSpec((1, D, F), lambda i, te: (te[i], 0, 0)),
            pl.BlockSpec((1, D, F), lambda i, te: (te[i], 0, 0)),
            pl.BlockSpec((1, F, D), lambda i, te: (te[i], 0, 0)),
            pl.BlockSpec((1, 1, TILE), lambda i, te: (i, 0, 0)),
        ],
        out_specs=pl.BlockSpec((TILE, D), lambda i, te: (i, 0)),
    )
    return pl.pallas_call(
        _ffn_body,
        grid_spec=grid_spec,
        out_shape=jax.ShapeDtypeStruct((P, D), jnp.float32),
    )(tile_expert, xs, gate_proj, up_proj, down_proj, sw3)


def _combine(ys, s1, s2):
    mesh = plsc.VectorSubcoreMesh(core_axis_name="c", subcore_axis_name="s")
    tok_w = T // NW
    nch = tok_w // CHUNK
    nvec = CHUNK * (D // LANES)

    @functools.partial(
        pl.kernel,
        out_type=jax.ShapeDtypeStruct((T, D), jnp.float32),
        mesh=mesh,
        scratch_types=[
            pltpu.VMEM((CHUNK,), jnp.int32),
            pltpu.VMEM((CHUNK,), jnp.int32),
            pltpu.VMEM((CHUNK, D), jnp.float32),
            pltpu.VMEM((CHUNK, D), jnp.float32),
            pltpu.SemaphoreType.DMA,
        ],
    )
    def k(s1_hbm, s2_hbm, ys_hbm, out_hbm, i1_v, i2_v, y1_v, y2_v, sem):
        wid = lax.axis_index("s") * 2 + lax.axis_index("c")
        base = wid * tok_w

        def chunk_body(c, carry):
            b = base + c * CHUNK
            pltpu.sync_copy(s1_hbm.at[pl.ds(b, CHUNK)], i1_v)
            pltpu.sync_copy(s2_hbm.at[pl.ds(b, CHUNK)], i2_v)
            pltpu.async_copy(ys_hbm.at[i1_v], y1_v, sem).wait()
            pltpu.async_copy(ys_hbm.at[i2_v], y2_v, sem).wait()

            def add_body(i, carry2):
                r = i // (D // LANES)
                col = (i % (D // LANES)) * LANES
                a = y1_v[r, pl.ds(col, LANES)]
                bvec = y2_v[r, pl.ds(col, LANES)]
                y1_v[r, pl.ds(col, LANES)] = a + bvec
                return carry2

            lax.fori_loop(0, nvec, add_body, 0)
            pltpu.sync_copy(y1_v, out_hbm.at[pl.ds(b, CHUNK)])
            return carry

        lax.fori_loop(0, nch, chunk_body, 0)

    return k(s1, s2, ys)


def kernel(x, router_kernel, gate_proj, up_proj, down_proj):
    b, t, d = x.shape
    xf = x.reshape(t, d)
    i1, i2, w1, w2, xpacked = _router(xf, router_kernel)
    sorted_tok, sorted_w, tile_expert, s1, s2 = _build_metadata(i1, i2, w1, w2)
    xs_packed = _dispatch(xpacked, sorted_tok)
    ys = _ffn(xs_packed, gate_proj, up_proj, down_proj,
              sorted_w, tile_expert)
    out = _combine(ys, s1, s2)
    return out.reshape(b, t, d)


# speedup vs baseline: 1.2861x; 1.0011x over previous
import functools

import jax
import jax.numpy as jnp
from jax import lax
from jax.experimental import pallas as pl
from jax.experimental.pallas import tpu as pltpu
from jax.experimental.pallas import tpu_sc as plsc

T, D, F, E, K = 2048, 1024, 768, 64, 2
N = T * K
TILE = 64
P = 8192
P_TILES = P // TILE
NW = 32
CHUNK = 32
LANES = 16


def _rtne_bf16_bits(v):
    bits = lax.bitcast_convert_type(v, jnp.int32)
    lsb = lax.shift_right_logical(bits, 16) & 1
    return (bits + 0x7FFF + lsb) & jnp.int32(-65536)


def _router_body(x_ref, rk_ref, i1_ref, i2_ref, w1_ref, w2_ref, xp_ref):
    xv = x_ref[...]
    ha = _rtne_bf16_bits(xv[:, : D // 2])
    hb = _rtne_bf16_bits(xv[:, D // 2:])
    xp_ref[...] = ha | lax.shift_right_logical(hb, 16)
    s = jnp.dot(x_ref[...], rk_ref[...], preferred_element_type=jnp.float32)
    iota = lax.broadcasted_iota(jnp.int32, s.shape, 1)
    m1 = jnp.max(s, axis=1, keepdims=True)
    i1 = jnp.min(jnp.where(s == m1, iota, E), axis=1, keepdims=True)
    s2 = jnp.where(iota == i1, -jnp.inf, s)
    m2 = jnp.max(s2, axis=1, keepdims=True)
    i2 = jnp.min(jnp.where(s2 == m2, iota, E), axis=1, keepdims=True)
    w1 = 1.0 / (1.0 + jnp.exp(m2 - m1))
    i1_ref[...] = i1
    i2_ref[...] = i2
    w1_ref[...] = w1
    w2_ref[...] = 1.0 - w1


def _router(xf, rk):
    i1, i2, w1, w2, xbf = pl.pallas_call(
        _router_body,
        out_shape=[
            jax.ShapeDtypeStruct((T, 1), jnp.int32),
            jax.ShapeDtypeStruct((T, 1), jnp.int32),
            jax.ShapeDtypeStruct((T, 1), jnp.float32),
            jax.ShapeDtypeStruct((T, 1), jnp.float32),
            jax.ShapeDtypeStruct((T, D // 2), jnp.int32),
        ],
    )(xf, rk)
    return i1[:, 0], i2[:, 0], w1[:, 0], w2[:, 0], xbf


def _build_metadata(i1, i2, w1, w2):
    e_pairs = jnp.stack([i1, i2], axis=1).reshape(-1)
    w_pairs = jnp.stack([w1, w2], axis=1).reshape(-1)
    oh = (e_pairs[:, None] == jnp.arange(E)[None, :]).astype(jnp.int32)
    csum = jnp.cumsum(oh, axis=0)
    rank = jnp.sum(csum * oh, axis=1) - 1
    counts = csum[-1]
    padded = ((counts + TILE - 1) // TILE) * TILE
    starts = jnp.concatenate([jnp.zeros((1,), jnp.int32),
                              jnp.cumsum(padded)[:-1].astype(jnp.int32)])
    slot = (starts[e_pairs] + rank).astype(jnp.int32)
    tile_cum = jnp.cumsum(padded // TILE)
    tile_expert = jnp.minimum(
        jnp.searchsorted(tile_cum, jnp.arange(P_TILES), side="right"),
        E - 1).astype(jnp.int32)
    tok_of_pair = (jnp.arange(N, dtype=jnp.int32) // K)
    sorted_tok = jnp.zeros((P,), jnp.int32).at[slot].set(tok_of_pair)
    sorted_w = jnp.zeros((P,), jnp.float32).at[slot].set(w_pairs)
    sp = slot.reshape(T, K)
    return sorted_tok, sorted_w, tile_expert, sp[:, 0], sp[:, 1]


def _dispatch(xpacked, sorted_tok):
    mesh = plsc.VectorSubcoreMesh(core_axis_name="c", subcore_axis_name="s")
    rows_w = P // NW
    dch = 128
    nch = rows_w // dch
    tok3 = sorted_tok.reshape(NW, nch, dch)

    @functools.partial(
        pl.kernel,
        out_type=jax.ShapeDtypeStruct((P, D // 2), jnp.int32),
        mesh=mesh,
        scratch_types=[
            pltpu.VMEM((nch, dch), jnp.int32),
            pltpu.VMEM((dch, D // 2), jnp.int32),
            pltpu.SemaphoreType.DMA,
            pltpu.SemaphoreType.DMA,
        ],
    )
    def k(tok_hbm, x_hbm, xs_hbm, idx_v, buf, sem_g, sem_o):
        wid = lax.axis_index("s") * 2 + lax.axis_index("c")
        base = wid * rows_w
        pltpu.sync_copy(tok_hbm.at[wid], idx_v)
        for c in range(nch):
            pltpu.async_copy(x_hbm.at[idx_v.at[c]], buf, sem_g).wait()
            pltpu.async_copy(
                buf, xs_hbm.at[pl.ds(base + c * dch, dch)], sem_o).wait()

    return k(tok3, xpacked)


def _ffn_body(te_ref, xs_ref, g_ref, u_ref, d_ref, sw_ref, ys_ref):
    packed = xs_ref[...]
    xa = lax.bitcast_convert_type(packed & jnp.int32(-65536), jnp.float32)
    xb = lax.bitcast_convert_type(lax.shift_left(packed, 16), jnp.float32)
    xt = jnp.concatenate([xa, xb], axis=1)
    g = jnp.dot(xt, g_ref[0], preferred_element_type=jnp.float32)
    u = jnp.dot(xt, u_ref[0], preferred_element_type=jnp.float32)
    h = g * jax.nn.sigmoid(g) * u
    y = jnp.dot(h, d_ref[0], preferred_element_type=jnp.float32)
    ys_ref[...] = y * sw_ref[0, 0, :][:, None]


def _ffn(xs, gate_proj, up_proj, down_proj, sorted_w, tile_expert):
    sw3 = sorted_w.reshape(P_TILES, 1, TILE)
    grid_spec = pltpu.PrefetchScalarGridSpec(
        num_scalar_prefetch=1,
        grid=(P_TILES,),
        in_specs=[
            pl.BlockSpec((TILE, D // 2), lambda i, te: (i, 0)),
            pl.BlockSpec((1, D, F), lambda i, te: (te[i], 0, 0)),
            pl.BlockSpec((1, D, F), lambda i, te: (te[i], 0, 0)),
            pl.BlockSpec((1, F, D), lambda i, te: (te[i], 0, 0)),
            pl.BlockSpec((1, 1, TILE), lambda i, te: (i, 0, 0)),
        ],
        out_specs=pl.BlockSpec((TILE, D), lambda i, te: (i, 0)),
    )
    return pl.pallas_call(
        _ffn_body,
        grid_spec=grid_spec,
        out_shape=jax.ShapeDtypeStruct((P, D), jnp.float32),
    )(tile_expert, xs, gate_proj, up_proj, down_proj, sw3)


def _combine(ys, s1, s2):
    mesh = plsc.VectorSubcoreMesh(core_axis_name="c", subcore_axis_name="s")
    tok_w = T // NW
    nch = tok_w // CHUNK
    nvec = CHUNK * (D // LANES)

    @functools.partial(
        pl.kernel,
        out_type=jax.ShapeDtypeStruct((T, D), jnp.float32),
        mesh=mesh,
        scratch_types=[
            pltpu.VMEM((CHUNK,), jnp.int32),
            pltpu.VMEM((CHUNK,), jnp.int32),
            pltpu.VMEM((CHUNK, D), jnp.float32),
            pltpu.VMEM((CHUNK, D), jnp.float32),
            pltpu.SemaphoreType.DMA,
        ],
    )
    def k(s1_hbm, s2_hbm, ys_hbm, out_hbm, i1_v, i2_v, y1_v, y2_v, sem):
        wid = lax.axis_index("s") * 2 + lax.axis_index("c")
        base = wid * tok_w

        def chunk_body(c, carry):
            b = base + c * CHUNK
            pltpu.sync_copy(s1_hbm.at[pl.ds(b, CHUNK)], i1_v)
            pltpu.sync_copy(s2_hbm.at[pl.ds(b, CHUNK)], i2_v)
            pltpu.async_copy(ys_hbm.at[i1_v], y1_v, sem).wait()
            pltpu.async_copy(ys_hbm.at[i2_v], y2_v, sem).wait()

            def add_body(i, carry2):
                r = i // (D // LANES)
                col = (i % (D // LANES)) * LANES
                a = y1_v[r, pl.ds(col, LANES)]
                bvec = y2_v[r, pl.ds(col, LANES)]
                y1_v[r, pl.ds(col, LANES)] = a + bvec
                return carry2

            lax.fori_loop(0, nvec, add_body, 0)
            pltpu.sync_copy(y1_v, out_hbm.at[pl.ds(b, CHUNK)])
            return carry

        lax.fori_loop(0, nch, chunk_body, 0)

    return k(s1, s2, ys)


def kernel(x, router_kernel, gate_proj, up_proj, down_proj):
    b, t, d = x.shape
    xf = x.reshape(t, d)
    i1, i2, w1, w2, xpacked = _router(xf, router_kernel)
    sorted_tok, sorted_w, tile_expert, s1, s2 = _build_metadata(i1, i2, w1, w2)
    xs_packed = _dispatch(xpacked, sorted_tok)
    ys = _ffn(xs_packed, gate_proj, up_proj, down_proj,
              sorted_w, tile_expert)
    out = _combine(ys, s1, s2)
    return out.reshape(b, t, d)


# speedup vs baseline: 2.2233x; 1.7287x over previous
import functools

import jax
import jax.numpy as jnp
from jax import lax
from jax.experimental import pallas as pl
from jax.experimental.pallas import tpu as pltpu
from jax.experimental.pallas import tpu_sc as plsc

T, D, F, E, K = 2048, 1024, 768, 64, 2
N = T * K
TILE = 64
P = 8192
P_TILES = P // TILE
NW = 32
LANES = 16
WL = 128
RB = 128
NB = T // RB


def _rtne_bf16_bits(v):
    bits = lax.bitcast_convert_type(v, jnp.int32)
    lsb = lax.shift_right_logical(bits, 16) & 1
    return (bits + 0x7FFF + lsb) & jnp.int32(-65536)


def _route_body(x_ref, rk_ref, s1_ref, s2_ref, w1_ref, w2_ref, te_ref,
                xp_ref):
    xv = x_ref[...]
    ha = _rtne_bf16_bits(xv[:, : D // 2])
    hb = _rtne_bf16_bits(xv[:, D // 2:])
    xp_ref[...] = ha | lax.shift_right_logical(hb, 16)

    s = jnp.dot(xv, rk_ref[...], preferred_element_type=jnp.float32)
    iota = lax.broadcasted_iota(jnp.int32, (T, E), 1)
    m1 = jnp.max(s, axis=1, keepdims=True)
    i1 = jnp.min(jnp.where(s == m1, iota, E), axis=1, keepdims=True)
    sm = jnp.where(iota == i1, -jnp.inf, s)
    m2 = jnp.max(sm, axis=1, keepdims=True)
    i2 = jnp.min(jnp.where(sm == m2, iota, E), axis=1, keepdims=True)
    w1 = 1.0 / (1.0 + jnp.exp(m2 - m1))
    w1_ref[...] = jnp.broadcast_to(w1, (T, WL))
    w2_ref[...] = jnp.broadcast_to(1.0 - w1, (T, WL))

    oh1 = (iota == i1).astype(jnp.float32)
    oh2 = (iota == i2).astype(jnp.float32)
    a = oh1 + oh2
    li = lax.broadcasted_iota(jnp.int32, (RB, RB), 0)
    lj = lax.broadcasted_iota(jnp.int32, (RB, RB), 1)
    ltri = (lj < li).astype(jnp.float32)
    bsums = []
    sx_blocks = []
    for b in range(NB):
        ab = a[b * RB:(b + 1) * RB, :]
        sx_blocks.append(jnp.dot(ltri, ab, preferred_element_type=jnp.float32))
        bsums.append(jnp.sum(ab, axis=0, keepdims=True))
    bs = jnp.concatenate(bsums, axis=0)
    ci = lax.broadcasted_iota(jnp.int32, (NB, NB), 0)
    cj = lax.broadcasted_iota(jnp.int32, (NB, NB), 1)
    ctri = (cj < ci).astype(jnp.float32)
    carry = jnp.dot(ctri, bs, preferred_element_type=jnp.float32)

    counts = jnp.sum(bs, axis=0, keepdims=True)
    tilecnt = lax.shift_right_logical(
        counts.astype(jnp.int32) + (TILE - 1), 6).astype(jnp.float32)
    ui = lax.broadcasted_iota(jnp.int32, (E, E), 0)
    uj = lax.broadcasted_iota(jnp.int32, (E, E), 1)
    uex = (ui < uj).astype(jnp.float32)
    uin = (ui <= uj).astype(jnp.float32)
    starts = jnp.dot(tilecnt, uex,
                     preferred_element_type=jnp.float32) * float(TILE)
    tcum = jnp.dot(tilecnt, uin, preferred_element_type=jnp.float32)

    r1_blocks = []
    r2_blocks = []
    for b in range(NB):
        sx = sx_blocks[b] + carry[b:b + 1, :]
        o1 = oh1[b * RB:(b + 1) * RB, :]
        o2 = oh2[b * RB:(b + 1) * RB, :]
        r1_blocks.append(jnp.sum((sx + starts) * o1, axis=1, keepdims=True))
        r2_blocks.append(jnp.sum((sx + starts) * o2, axis=1, keepdims=True))
    s1_ref[...] = jnp.concatenate(r1_blocks, axis=0).astype(jnp.int32)
    s2_ref[...] = (jnp.concatenate(r2_blocks, axis=0)
                   + jnp.sum(oh1 * oh2, axis=1, keepdims=True)
                   ).astype(jnp.int32)

    jt = lax.broadcasted_iota(jnp.int32, (P_TILES, E), 0)
    te = jnp.sum((tcum.astype(jnp.int32) <= jt).astype(jnp.int32),
                 axis=1, keepdims=True)
    te_ref[...] = jnp.minimum(te, E - 1)


def _route(xf, rk):
    return pl.pallas_call(
        _route_body,
        out_shape=[
            jax.ShapeDtypeStruct((T, 1), jnp.int32),
            jax.ShapeDtypeStruct((T, 1), jnp.int32),
            jax.ShapeDtypeStruct((T, WL), jnp.float32),
            jax.ShapeDtypeStruct((T, WL), jnp.float32),
            jax.ShapeDtypeStruct((P_TILES, 1), jnp.int32),
            jax.ShapeDtypeStruct((T, D // 2), jnp.int32),
        ],
    )(xf, rk)


def _dispatch(xpacked, s1, s2, w1, w2):
    mesh = plsc.VectorSubcoreMesh(core_axis_name="c", subcore_axis_name="s")
    tok_w = T // NW
    s1r = s1.reshape(NW, tok_w)
    s2r = s2.reshape(NW, tok_w)
    w1r = w1.reshape(NW, tok_w, WL)
    w2r = w2.reshape(NW, tok_w, WL)

    @functools.partial(
        pl.kernel,
        out_type=[
            jax.ShapeDtypeStruct((P, D // 2), jnp.int32),
            jax.ShapeDtypeStruct((P, WL), jnp.float32),
        ],
        mesh=mesh,
        scratch_types=[
            pltpu.VMEM((tok_w,), jnp.int32),
            pltpu.VMEM((tok_w,), jnp.int32),
            pltpu.VMEM((tok_w, D // 2), jnp.int32),
            pltpu.VMEM((tok_w, WL), jnp.float32),
            pltpu.VMEM((tok_w, WL), jnp.float32),
            pltpu.SemaphoreType.DMA,
            pltpu.SemaphoreType.DMA,
        ],
    )
    def k(s1_hbm, s2_hbm, w1_hbm, w2_hbm, x_hbm, xs_hbm, sw_hbm,
          i1_v, i2_v, rows_v, wa_v, wb_v, sem_i, sem_d):
        wid = lax.axis_index("s") * 2 + lax.axis_index("c")
        base = wid * tok_w
        c1 = pltpu.async_copy(s1_hbm.at[wid], i1_v, sem_i)
        c2 = pltpu.async_copy(s2_hbm.at[wid], i2_v, sem_i)
        c3 = pltpu.async_copy(w1_hbm.at[wid], wa_v, sem_i)
        c4 = pltpu.async_copy(w2_hbm.at[wid], wb_v, sem_i)
        c5 = pltpu.async_copy(x_hbm.at[pl.ds(base, tok_w)], rows_v, sem_d)
        c1.wait()
        c2.wait()
        c3.wait()
        c4.wait()
        c5.wait()
        o1 = pltpu.async_copy(rows_v, xs_hbm.at[i1_v], sem_d)
        o2 = pltpu.async_copy(rows_v, xs_hbm.at[i2_v], sem_d)
        o3 = pltpu.async_copy(wa_v, sw_hbm.at[i1_v], sem_i)
        o4 = pltpu.async_copy(wb_v, sw_hbm.at[i2_v], sem_i)
        o1.wait()
        o2.wait()
        o3.wait()
        o4.wait()

    return k(s1r, s2r, w1r, w2r, xpacked)


def _ffn_body(te_ref, xs_ref, g_ref, u_ref, d_ref, sw_ref, ys_ref):
    packed = xs_ref[...]
    xa = lax.bitcast_convert_type(packed & jnp.int32(-65536), jnp.float32)
    xb = lax.bitcast_convert_type(lax.shift_left(packed, 16), jnp.float32)
    xt = jnp.concatenate([xa, xb], axis=1)
    g = jnp.dot(xt, g_ref[0], preferred_element_type=jnp.float32)
    u = jnp.dot(xt, u_ref[0], preferred_element_type=jnp.float32)
    h = g * jax.nn.sigmoid(g) * u
    y = jnp.dot(h, d_ref[0], preferred_element_type=jnp.float32)
    ys_ref[...] = y * sw_ref[...][:, 0:1]


def _ffn(xs, gate_proj, up_proj, down_proj, sw, tile_expert):
    grid_spec = pltpu.PrefetchScalarGridSpec(
        num_scalar_prefetch=1,
        grid=(P_TILES,),
        in_specs=[
            pl.BlockSpec((TILE, D // 2), lambda i, te: (i, 0)),
            pl.BlockSpec((1, D, F), lambda i, te: (te[i], 0, 0)),
            pl.BlockSpec((1, D, F), lambda i, te: (te[i], 0, 0)),
            pl.BlockSpec((1, F, D), lambda i, te: (te[i], 0, 0)),
            pl.BlockSpec((TILE, WL), lambda i, te: (i, 0)),
        ],
        out_specs=pl.BlockSpec((TILE, D), lambda i, te: (i, 0)),
    )
    return pl.pallas_call(
        _ffn_body,
        grid_spec=grid_spec,
        out_shape=jax.ShapeDtypeStruct((P, D), jnp.float32),
    )(tile_expert, xs, gate_proj, up_proj, down_proj, sw)


def _combine(ys, s1, s2):
    mesh = plsc.VectorSubcoreMesh(core_axis_name="c", subcore_axis_name="s")
    tok_w = T // NW
    chunk = 32
    nch = tok_w // chunk
    s1r = s1.reshape(NW, nch, chunk)
    s2r = s2.reshape(NW, nch, chunk)

    @functools.partial(
        pl.kernel,
        out_type=jax.ShapeDtypeStruct((T, D), jnp.float32),
        mesh=mesh,
        scratch_types=[
            pltpu.VMEM((chunk,), jnp.int32),
            pltpu.VMEM((chunk,), jnp.int32),
            pltpu.VMEM((chunk, D), jnp.float32),
            pltpu.VMEM((chunk, D), jnp.float32),
            pltpu.SemaphoreType.DMA,
            pltpu.SemaphoreType.DMA,
        ],
    )
    def k(s1_hbm, s2_hbm, ys_hbm, out_hbm,
          i1_v, i2_v, y1_v, y2_v, sem_i, sem_d):
        wid = lax.axis_index("s") * 2 + lax.axis_index("c")
        base = wid * tok_w

        def chunk_body(c, carry):
            b = base + c * chunk
            pltpu.async_copy(s1_hbm.at[wid, c], i1_v, sem_i).wait()
            pltpu.async_copy(s2_hbm.at[wid, c], i2_v, sem_i).wait()
            g1 = pltpu.async_copy(ys_hbm.at[i1_v], y1_v, sem_d)
            g2 = pltpu.async_copy(ys_hbm.at[i2_v], y2_v, sem_d)
            g1.wait()
            g2.wait()

            def row_body(r, carry2):
                def col_body(cc, carry3):
                    sl = pl.ds(cc * LANES, LANES)
                    y1_v[r, sl] = y1_v[r, sl] + y2_v[r, sl]
                    return carry3

                lax.fori_loop(0, D // LANES, col_body, 0)
                return carry2

            lax.fori_loop(0, chunk, row_body, 0)
            pltpu.sync_copy(y1_v, out_hbm.at[pl.ds(b, chunk)])
            return carry

        lax.fori_loop(0, nch, chunk_body, 0)

    return k(s1r, s2r, ys)


def kernel(x, router_kernel, gate_proj, up_proj, down_proj):
    b, t, d = x.shape
    xf = x.reshape(t, d)
    s1, s2, w1, w2, te, xpacked = _route(xf, router_kernel)
    s1 = s1[:, 0]
    s2 = s2[:, 0]
    xs_packed, sw = _dispatch(xpacked, s1, s2, w1, w2)
    ys = _ffn(xs_packed, gate_proj, up_proj, down_proj, sw, te[:, 0])
    out = _combine(ys, s1, s2)
    return out.reshape(b, t, d)


# speedup vs baseline: 2.2556x; 1.0145x over previous
import functools

import jax
import jax.numpy as jnp
from jax import lax
from jax.experimental import pallas as pl
from jax.experimental.pallas import tpu as pltpu
from jax.experimental.pallas import tpu_sc as plsc

T, D, F, E, K = 2048, 1024, 768, 64, 2
N = T * K
TILE = 64
P = 8192
P_TILES = P // TILE
NW = 32
LANES = 16
WL = 128
RB = 128
NB = T // RB


def _rtne_bf16_bits(v):
    bits = lax.bitcast_convert_type(v, jnp.int32)
    lsb = lax.shift_right_logical(bits, 16) & 1
    return (bits + 0x7FFF + lsb) & jnp.int32(-65536)


def _route_body(x_ref, rk_ref, s1_ref, s2_ref, w1_ref, w2_ref, te_ref,
                xp_ref):
    xv = x_ref[...]
    ha = _rtne_bf16_bits(xv[:, : D // 2])
    hb = _rtne_bf16_bits(xv[:, D // 2:])
    xp_ref[...] = ha | lax.shift_right_logical(hb, 16)

    s = jnp.dot(xv, rk_ref[...], preferred_element_type=jnp.float32)
    iota = lax.broadcasted_iota(jnp.int32, (T, E), 1)
    m1 = jnp.max(s, axis=1, keepdims=True)
    i1 = jnp.min(jnp.where(s == m1, iota, E), axis=1, keepdims=True)
    sm = jnp.where(iota == i1, -jnp.inf, s)
    m2 = jnp.max(sm, axis=1, keepdims=True)
    i2 = jnp.min(jnp.where(sm == m2, iota, E), axis=1, keepdims=True)
    w1 = 1.0 / (1.0 + jnp.exp(m2 - m1))
    w1_ref[...] = jnp.broadcast_to(w1, (T, WL))
    w2_ref[...] = jnp.broadcast_to(1.0 - w1, (T, WL))

    oh1 = (iota == i1).astype(jnp.float32)
    oh2 = (iota == i2).astype(jnp.float32)
    a = oh1 + oh2
    li = lax.broadcasted_iota(jnp.int32, (RB, RB), 0)
    lj = lax.broadcasted_iota(jnp.int32, (RB, RB), 1)
    ltri = (lj < li).astype(jnp.float32)
    bsums = []
    sx_blocks = []
    for b in range(NB):
        ab = a[b * RB:(b + 1) * RB, :]
        sx_blocks.append(jnp.dot(ltri, ab, preferred_element_type=jnp.float32))
        bsums.append(jnp.sum(ab, axis=0, keepdims=True))
    bs = jnp.concatenate(bsums, axis=0)
    ci = lax.broadcasted_iota(jnp.int32, (NB, NB), 0)
    cj = lax.broadcasted_iota(jnp.int32, (NB, NB), 1)
    ctri = (cj < ci).astype(jnp.float32)
    carry = jnp.dot(ctri, bs, preferred_element_type=jnp.float32)

    counts = jnp.sum(bs, axis=0, keepdims=True)
    tilecnt = lax.shift_right_logical(
        counts.astype(jnp.int32) + (TILE - 1), 6).astype(jnp.float32)
    ui = lax.broadcasted_iota(jnp.int32, (E, E), 0)
    uj = lax.broadcasted_iota(jnp.int32, (E, E), 1)
    uex = (ui < uj).astype(jnp.float32)
    uin = (ui <= uj).astype(jnp.float32)
    starts = jnp.dot(tilecnt, uex,
                     preferred_element_type=jnp.float32) * float(TILE)
    tcum = jnp.dot(tilecnt, uin, preferred_element_type=jnp.float32)

    r1_blocks = []
    r2_blocks = []
    for b in range(NB):
        sx = sx_blocks[b] + carry[b:b + 1, :]
        o1 = oh1[b * RB:(b + 1) * RB, :]
        o2 = oh2[b * RB:(b + 1) * RB, :]
        r1_blocks.append(jnp.sum((sx + starts) * o1, axis=1, keepdims=True))
        r2_blocks.append(jnp.sum((sx + starts) * o2, axis=1, keepdims=True))
    s1_ref[...] = jnp.concatenate(r1_blocks, axis=0).astype(jnp.int32)
    s2_ref[...] = (jnp.concatenate(r2_blocks, axis=0)
                   + jnp.sum(oh1 * oh2, axis=1, keepdims=True)
                   ).astype(jnp.int32)

    jt = lax.broadcasted_iota(jnp.int32, (P_TILES, E), 0)
    te = jnp.sum((tcum.astype(jnp.int32) <= jt).astype(jnp.int32),
                 axis=1, keepdims=True)
    te_ref[...] = jnp.minimum(te, E - 1)


def _route(xf, rk):
    return pl.pallas_call(
        _route_body,
        out_shape=[
            jax.ShapeDtypeStruct((T, 1), jnp.int32),
            jax.ShapeDtypeStruct((T, 1), jnp.int32),
            jax.ShapeDtypeStruct((T, WL), jnp.float32),
            jax.ShapeDtypeStruct((T, WL), jnp.float32),
            jax.ShapeDtypeStruct((P_TILES, 1), jnp.int32),
            jax.ShapeDtypeStruct((T, D // 2), jnp.int32),
        ],
    )(xf, rk)


def _dispatch(xpacked, s1, s2, w1, w2):
    mesh = plsc.VectorSubcoreMesh(core_axis_name="c", subcore_axis_name="s")
    tok_w = T // NW
    s1r = s1.reshape(NW, tok_w)
    s2r = s2.reshape(NW, tok_w)
    w1r = w1.reshape(NW, tok_w, WL)
    w2r = w2.reshape(NW, tok_w, WL)

    @functools.partial(
        pl.kernel,
        out_type=[
            jax.ShapeDtypeStruct((P, D // 2), jnp.int32),
            jax.ShapeDtypeStruct((P, WL), jnp.float32),
        ],
        mesh=mesh,
        scratch_types=[
            pltpu.VMEM((tok_w,), jnp.int32),
            pltpu.VMEM((tok_w,), jnp.int32),
            pltpu.VMEM((tok_w, D // 2), jnp.int32),
            pltpu.VMEM((tok_w, WL), jnp.float32),
            pltpu.VMEM((tok_w, WL), jnp.float32),
            pltpu.SemaphoreType.DMA,
            pltpu.SemaphoreType.DMA,
        ],
    )
    def k(s1_hbm, s2_hbm, w1_hbm, w2_hbm, x_hbm, xs_hbm, sw_hbm,
          i1_v, i2_v, rows_v, wa_v, wb_v, sem_i, sem_d):
        wid = lax.axis_index("s") * 2 + lax.axis_index("c")
        base = wid * tok_w
        c1 = pltpu.async_copy(s1_hbm.at[wid], i1_v, sem_i)
        c2 = pltpu.async_copy(s2_hbm.at[wid], i2_v, sem_i)
        c3 = pltpu.async_copy(w1_hbm.at[wid], wa_v, sem_i)
        c4 = pltpu.async_copy(w2_hbm.at[wid], wb_v, sem_i)
        c5 = pltpu.async_copy(x_hbm.at[pl.ds(base, tok_w)], rows_v, sem_d)
        c1.wait()
        c2.wait()
        c3.wait()
        c4.wait()
        c5.wait()
        o1 = pltpu.async_copy(rows_v, xs_hbm.at[i1_v], sem_d)
        o2 = pltpu.async_copy(rows_v, xs_hbm.at[i2_v], sem_d)
        o3 = pltpu.async_copy(wa_v, sw_hbm.at[i1_v], sem_i)
        o4 = pltpu.async_copy(wb_v, sw_hbm.at[i2_v], sem_i)
        o1.wait()
        o2.wait()
        o3.wait()
        o4.wait()

    return k(s1r, s2r, w1r, w2r, xpacked)


def _ffn_body(te_ref, xs_ref, g_ref, u_ref, d_ref, sw_ref, ys_ref):
    packed = xs_ref[...]
    xa = lax.bitcast_convert_type(packed & jnp.int32(-65536), jnp.float32)
    xb = lax.bitcast_convert_type(lax.shift_left(packed, 16), jnp.float32)
    xt = jnp.concatenate([xa, xb], axis=1)
    g = jnp.dot(xt, g_ref[0], preferred_element_type=jnp.float32)
    u = jnp.dot(xt, u_ref[0], preferred_element_type=jnp.float32)
    h = g * jax.nn.sigmoid(g) * u
    y = jnp.dot(h, d_ref[0], preferred_element_type=jnp.float32)
    ys_ref[...] = y * sw_ref[...][:, 0:1]


def _ffn(xs, gate_proj, up_proj, down_proj, sw, tile_expert):
    grid_spec = pltpu.PrefetchScalarGridSpec(
        num_scalar_prefetch=1,
        grid=(P_TILES,),
        in_specs=[
            pl.BlockSpec((TILE, D // 2), lambda i, te: (i, 0)),
            pl.BlockSpec((1, D, F), lambda i, te: (te[i], 0, 0)),
            pl.BlockSpec((1, D, F), lambda i, te: (te[i], 0, 0)),
            pl.BlockSpec((1, F, D), lambda i, te: (te[i], 0, 0)),
            pl.BlockSpec((TILE, WL), lambda i, te: (i, 0)),
        ],
        out_specs=pl.BlockSpec((TILE, D), lambda i, te: (i, 0)),
    )
    return pl.pallas_call(
        _ffn_body,
        grid_spec=grid_spec,
        out_shape=jax.ShapeDtypeStruct((P, D), jnp.float32),
    )(tile_expert, xs, gate_proj, up_proj, down_proj, sw)


def _combine(ys, s1, s2):
    mesh = plsc.VectorSubcoreMesh(core_axis_name="c", subcore_axis_name="s")
    tok_w = T // NW
    chunk = 16
    nch = tok_w // chunk
    s1r = s1.reshape(NW, nch, chunk)
    s2r = s2.reshape(NW, nch, chunk)

    @functools.partial(
        pl.kernel,
        out_type=jax.ShapeDtypeStruct((T, D), jnp.float32),
        mesh=mesh,
        scratch_types=[
            pltpu.VMEM((nch, chunk), jnp.int32),
            pltpu.VMEM((nch, chunk), jnp.int32),
            pltpu.VMEM((2, chunk, D), jnp.float32),
            pltpu.VMEM((2, chunk, D), jnp.float32),
            pltpu.SemaphoreType.DMA,
            pltpu.SemaphoreType.DMA,
            pltpu.SemaphoreType.DMA,
        ],
    )
    def k(s1_hbm, s2_hbm, ys_hbm, out_hbm,
          i1_v, i2_v, y1_v, y2_v, sem_i, sem_d, sem_o):
        wid = lax.axis_index("s") * 2 + lax.axis_index("c")
        base = wid * tok_w
        ca = pltpu.async_copy(s1_hbm.at[wid], i1_v, sem_i)
        cb = pltpu.async_copy(s2_hbm.at[wid], i2_v, sem_i)
        ca.wait()
        cb.wait()
        pltpu.async_copy(ys_hbm.at[i1_v.at[0]], y1_v.at[0], sem_d)
        pltpu.async_copy(ys_hbm.at[i2_v.at[0]], y2_v.at[0], sem_d)
        for c in range(nch):
            r = c % 2
            pltpu.make_async_copy(
                ys_hbm.at[i1_v.at[c]], y1_v.at[r], sem_d).wait()
            pltpu.make_async_copy(
                ys_hbm.at[i2_v.at[c]], y2_v.at[r], sem_d).wait()
            if c >= 1:
                pltpu.make_async_copy(
                    y1_v.at[(c - 1) % 2],
                    out_hbm.at[pl.ds(base + (c - 1) * chunk, chunk)],
                    sem_o).wait()
            if c + 1 < nch:
                pltpu.async_copy(
                    ys_hbm.at[i1_v.at[c + 1]], y1_v.at[(c + 1) % 2], sem_d)
                pltpu.async_copy(
                    ys_hbm.at[i2_v.at[c + 1]], y2_v.at[(c + 1) % 2], sem_d)

            def row_body(rr, carry2, _r=r):
                def col_body(cc, carry3):
                    sl = pl.ds(cc * LANES, LANES)
                    y1_v[_r, rr, sl] = y1_v[_r, rr, sl] + y2_v[_r, rr, sl]
                    return carry3

                lax.fori_loop(0, D // LANES, col_body, 0)
                return carry2

            lax.fori_loop(0, chunk, row_body, 0)
            pltpu.async_copy(
                y1_v.at[r],
                out_hbm.at[pl.ds(base + c * chunk, chunk)], sem_o)
        pltpu.make_async_copy(
            y1_v.at[(nch - 1) % 2],
            out_hbm.at[pl.ds(base + (nch - 1) * chunk, chunk)],
            sem_o).wait()

    return k(s1r, s2r, ys)


def kernel(x, router_kernel, gate_proj, up_proj, down_proj):
    b, t, d = x.shape
    xf = x.reshape(t, d)
    s1, s2, w1, w2, te, xpacked = _route(xf, router_kernel)
    s1 = s1[:, 0]
    s2 = s2[:, 0]
    xs_packed, sw = _dispatch(xpacked, s1, s2, w1, w2)
    ys = _ffn(xs_packed, gate_proj, up_proj, down_proj, sw, te[:, 0])
    out = _combine(ys, s1, s2)
    return out.reshape(b, t, d)
